# baseline (speedup 1.0000x reference)
import functools

import jax
import jax.numpy as jnp
from jax import lax
from jax.experimental import pallas as pl
from jax.experimental.pallas import tpu as pltpu
from jax.experimental.pallas import tpu_sc as plsc

NC = 2
NS = 16
NW = NC * NS
LANES = 16

_N = 10000
_E = 320000
_H = 128
_L = 5
_CH = 128
_NCHUNK = _E // _CH
_JMAX = -(-_NCHUNK // NW)
_AROWS = _N
_ZR = 80
_ZNCHUNK = _N // _ZR
_ZQ = -(-_ZNCHUNK // NS)
_NV = 10240
_VNCHUNK = _NV // _CH
_VJ = -(-_VNCHUNK // NW)
_CW = 32
_REP = 64


def _zero_2d(ref, nrows, ncols):
    z = jnp.zeros((LANES,), jnp.float32)

    def body(i, _):
        for g in range(ncols // LANES):
            ref[i, pl.ds(g * LANES, LANES)] = z
        return 0

    lax.fori_loop(0, nrows, body, 0)


def _make_edge_pre():
    mesh = plsc.VectorSubcoreMesh(core_axis_name="c", subcore_axis_name="s")

    @functools.partial(
        pl.kernel,
        out_type=(
            jax.ShapeDtypeStruct((2, NC, _N, _H), jnp.float32),
            jax.ShapeDtypeStruct((_NV, _H), jnp.float32),
        ),
        mesh=mesh,
        scratch_types=[
            pltpu.VMEM_SHARED((_AROWS, _H), jnp.float32),
            pltpu.VMEM((2, _CH, _H), jnp.float32),
            pltpu.VMEM((2, _CH), jnp.int32),
            pltpu.VMEM((2, _CH), jnp.int32),
            pltpu.SemaphoreType.DMA((2,)),
            pltpu.SemaphoreType.DMA((2,)),
            pltpu.SemaphoreType.DMA((2,)),
        ],
    )
    def k(tab_hbm, dst_hbm, gidx_hbm, xemb_hbm, vemb_hbm, zeros_hbm,
          part_out, veb_out,
          acc, rows, gb, db, isem, gsem, ssem):
        core = lax.axis_index("c")
        sid = lax.axis_index("s")
        wid = sid * NC + core
        nj = (_NCHUNK - wid + NW - 1) // NW

        def phase_body(phase, _):
            g0 = phase * _E

            def start_idx(j, p):
                e0 = (wid + NW * j) * _CH
                pltpu.async_copy(gidx_hbm.at[pl.ds(g0 + e0, _CH)],
                                 gb.at[p], isem.at[p])
                pltpu.async_copy(dst_hbm.at[pl.ds(e0, _CH)],
                                 db.at[p], isem.at[p])

            def wait_idx(p):
                pltpu.make_async_copy(gidx_hbm.at[pl.ds(g0, _CH)], gb.at[p],
                                      isem.at[p]).wait()
                pltpu.make_async_copy(dst_hbm.at[pl.ds(0, _CH)], db.at[p],
                                      isem.at[p]).wait()

            def start_gather(p):
                pltpu.async_copy(tab_hbm.at[gb.at[p]], rows.at[p],
                                 gsem.at[p])

            def finish(p):
                pltpu.make_async_copy(tab_hbm.at[gb.at[p]], rows.at[p],
                                      gsem.at[p]).wait()
                pltpu.async_copy(rows.at[p], acc.at[db.at[p]], ssem.at[p],
                                 add=True)

            def wait_scat(p):
                pltpu.make_async_copy(rows.at[p], acc.at[db.at[p]],
                                      ssem.at[p]).wait()

            start_idx(0, 0)
            start_idx(1, 1)

            def zbody(q, _):
                zc = sid + NS * q

                @pl.when(zc < _ZNCHUNK)
                def _():
                    r0 = pl.multiple_of(zc * _ZR, _ZR)
                    pltpu.async_copy(zeros_hbm, acc.at[pl.ds(r0, _ZR)],
                                     ssem.at[0])

                return 0

            def zdrain(q, _):
                zc = sid + NS * q

                @pl.when(zc < _ZNCHUNK)
                def _():
                    r0 = pl.multiple_of(zc * _ZR, _ZR)
                    pltpu.make_async_copy(zeros_hbm, acc.at[pl.ds(r0, _ZR)],
                                          ssem.at[0]).wait()

                return 0

            lax.fori_loop(0, _ZQ, zbody, 0)
            wait_idx(0)
            start_gather(0)
            lax.fori_loop(0, _ZQ, zdrain, 0)
            plsc.subcore_barrier()

            def body(j, _):
                p = lax.rem(j, 2)
                q = 1 - p

                @pl.when(j + 1 < nj)
                def _():
                    wait_idx(q)

                    @pl.when(j >= 1)
                    def _():
                        wait_scat(q)

                    start_gather(q)

                finish(p)

                @pl.when(j + 2 < nj)
                def _():
                    start_idx(j + 2, p)

                return 0

            lax.fori_loop(0, nj, body, 0)
            wait_scat(lax.rem(nj - 1, 2))
            wait_scat(lax.rem(nj, 2))

            plsc.subcore_barrier()

            def cbody(q, _):
                zc = sid + NS * q

                @pl.when(zc < _ZNCHUNK)
                def _():
                    r0 = pl.multiple_of(zc * _ZR, _ZR)
                    bb = rows.at[0, pl.ds(0, _ZR)]
                    pltpu.sync_copy(acc.at[pl.ds(r0, _ZR)], bb)
                    pltpu.sync_copy(bb,
                                    part_out.at[phase, core, pl.ds(r0, _ZR)])

                return 0

            lax.fori_loop(0, _ZQ, cbody, 0)
            plsc.subcore_barrier()
            return 0

        lax.fori_loop(0, 2, phase_body, 0)

        def vbody(j, _):
            cid = wid + NW * j

            @pl.when(cid < _VNCHUNK)
            def _():
                pltpu.sync_copy(xemb_hbm.at[pl.ds(cid * _CH, _CH)],
                                gb.at[0])
                pltpu.async_copy(vemb_hbm.at[gb.at[0]], rows.at[0],
                                 gsem.at[0]).wait()
                pltpu.sync_copy(rows.at[0],
                                veb_out.at[pl.ds(cid * _CH, _CH)])

            return 0

        lax.fori_loop(0, _VJ, vbody, 0)

    return k


def _make_spmv():
    mesh = plsc.VectorSubcoreMesh(core_axis_name="c", subcore_axis_name="s")

    @functools.partial(
        pl.kernel,
        out_type=jax.ShapeDtypeStruct((NC, _N, _H), jnp.float32),
        mesh=mesh,
        scratch_types=[
            pltpu.VMEM_SHARED((_AROWS, _H), jnp.float32),
            pltpu.VMEM((2, _CH, _H), jnp.float32),
            pltpu.VMEM((2, _CH), jnp.int32),
            pltpu.VMEM((2, _CH), jnp.int32),
            pltpu.SemaphoreType.DMA((2,)),
            pltpu.SemaphoreType.DMA((2,)),
            pltpu.SemaphoreType.DMA((2,)),
        ],
    )
    def k(h_hbm, src_hbm, dst_hbm, zeros_hbm, part_out, acc, rows, gb, db,
          isem, gsem, ssem):
        core = lax.axis_index("c")
        sid = lax.axis_index("s")
        wid = sid * NC + core
        nj = (_NCHUNK - wid + NW - 1) // NW

        def start_idx(j, p):
            e0 = (wid + NW * j) * _CH
            pltpu.async_copy(src_hbm.at[pl.ds(e0, _CH)], gb.at[p],
                             isem.at[p])
            pltpu.async_copy(dst_hbm.at[pl.ds(e0, _CH)], db.at[p],
                             isem.at[p])

        def wait_idx(p):
            pltpu.make_async_copy(src_hbm.at[pl.ds(0, _CH)], gb.at[p],
                                  isem.at[p]).wait()
            pltpu.make_async_copy(dst_hbm.at[pl.ds(0, _CH)], db.at[p],
                                  isem.at[p]).wait()

        def start_gather(p):
            pltpu.async_copy(h_hbm.at[gb.at[p]], rows.at[p], gsem.at[p])

        def finish(p):
            pltpu.make_async_copy(h_hbm.at[gb.at[p]], rows.at[p],
                                  gsem.at[p]).wait()
            pltpu.async_copy(rows.at[p], acc.at[db.at[p]], ssem.at[p],
                             add=True)

        def wait_scat(p):
            pltpu.make_async_copy(rows.at[p], acc.at[db.at[p]],
                                  ssem.at[p]).wait()

        start_idx(0, 0)
        start_idx(1, 1)

        def zbody(q, _):
            zc = sid + NS * q

            @pl.when(zc < _ZNCHUNK)
            def _():
                r0 = pl.multiple_of(zc * _ZR, _ZR)
                pltpu.async_copy(zeros_hbm, acc.at[pl.ds(r0, _ZR)],
                                 ssem.at[0])

            return 0

        def zdrain(q, _):
            zc = sid + NS * q

            @pl.when(zc < _ZNCHUNK)
            def _():
                r0 = pl.multiple_of(zc * _ZR, _ZR)
                pltpu.make_async_copy(zeros_hbm, acc.at[pl.ds(r0, _ZR)],
                                      ssem.at[0]).wait()

            return 0

        lax.fori_loop(0, _ZQ, zbody, 0)
        wait_idx(0)
        start_gather(0)
        lax.fori_loop(0, _ZQ, zdrain, 0)
        plsc.subcore_barrier()

        def body(j, _):
            p = lax.rem(j, 2)
            q = 1 - p

            @pl.when(j + 1 < nj)
            def _():
                wait_idx(q)

                @pl.when(j >= 1)
                def _():
                    wait_scat(q)

                start_gather(q)

            finish(p)

            @pl.when(j + 2 < nj)
            def _():
                start_idx(j + 2, p)

            return 0

        lax.fori_loop(0, nj, body, 0)
        wait_scat(lax.rem(nj - 1, 2))
        wait_scat(lax.rem(nj, 2))

        plsc.subcore_barrier()

        def cbody(q, _):
            zc = sid + NS * q

            @pl.when(zc < _ZNCHUNK)
            def _():
                r0 = pl.multiple_of(zc * _ZR, _ZR)
                bb = rows.at[0, pl.ds(0, _ZR)]
                pltpu.sync_copy(acc.at[pl.ds(r0, _ZR)], bb)
                pltpu.sync_copy(bb, part_out.at[core, pl.ds(r0, _ZR)])

            return 0

        lax.fori_loop(0, _ZQ, cbody, 0)

    return k


def _dot(a, b):
    ah = a.astype(jnp.bfloat16)
    al = (a - ah.astype(jnp.float32)).astype(jnp.bfloat16)
    bh = b.astype(jnp.bfloat16)
    bl = (b - bh.astype(jnp.float32)).astype(jnp.bfloat16)

    def d(u, v):
        return lax.dot_general(u, v, (((1,), (0,)), ((), ())),
                               preferred_element_type=jnp.float32)

    return d(ah, bh) + (d(ah, bl) + d(al, bh))


def _pre_body(x_ref, veb_ref, ap_ref, ep_ref, wv_ref, we_ref,
              h0_ref, eagg_ref):
    h0_ref[...] = _dot(x_ref[...], wv_ref[...]) + veb_ref[...]
    attr = ap_ref[0] + ap_ref[1]
    eagg_ref[...] = _dot(attr, we_ref[...]) + ep_ref[0] + ep_ref[1]


def _layer_common(h_ref, part_ref, eagg_ref, wlin_ref, wr_ref, wroot_ref,
                  b_ref, g_ref, be_ref, wres_ref):
    h = h_ref[...]
    agg = part_ref[0] + part_ref[1] + _dot(eagg_ref[...], wlin_ref[...])
    out = _dot(agg, wr_ref[...]) + _dot(h, wroot_ref[...]) + b_ref[...]
    out = jnp.maximum(out, 0.0)
    mean = jnp.mean(out, axis=0, keepdims=True)
    ctr = out - mean
    var = jnp.mean(ctr * ctr, axis=0, keepdims=True)
    outn = ctr * lax.rsqrt(var + 1e-5) * g_ref[...] + be_ref[...]
    return outn + _dot(h, wres_ref[...])


def _layer_body(h_ref, part_ref, eagg_ref, wlin_ref, wr_ref, wroot_ref,
                b_ref, g_ref, be_ref, wres_ref, out_ref):
    out_ref[...] = _layer_common(h_ref, part_ref, eagg_ref, wlin_ref, wr_ref,
                                 wroot_ref, b_ref, g_ref, be_ref, wres_ref)


def _head_body(h_ref, w1_ref, b1_ref, w2_ref, b2_ref, y_ref):
    t = _dot(h_ref[...], w1_ref[...]) + b1_ref[...]
    t = t * 0.5 * (1.0 + lax.erf(t * (2.0 ** -0.5)))
    y_ref[...] = _dot(t, w2_ref[...]) + b2_ref[...]


def kernel(x, x_emb, edge_index, edge_attr, edge_attr_emb, vert_ff_w,
           vert_emb_t, edge_ff_w, edge_emb_t, edge_lin_w, conv_rel_w,
           conv_root_w, conv_b, bn_gamma, bn_beta, res_lin_w, head_w1,
           head_b1, head_w2, head_b2):
    evoc = edge_emb_t.shape[0]
    src1d = edge_index[0]
    dst1d = edge_index[1]
    te_rep = jnp.tile(edge_emb_t, (_REP, 1))
    tab = jnp.concatenate([edge_attr, te_rep])
    eae_spread = (edge_attr_emb.astype(jnp.int32)
                  + evoc * (jnp.arange(_E, dtype=jnp.int32) % _REP))
    gidx = jnp.concatenate([
        jnp.arange(_E, dtype=jnp.int32),
        eae_spread + _E,
    ])
    xemb_pad = jnp.concatenate(
        [x_emb.astype(jnp.int32), jnp.zeros((_NV - _N,), jnp.int32)])
    zeros_blk = jnp.zeros((_ZR, _H), jnp.float32)

    part_pre, veb_pad = _make_edge_pre()(
        tab, dst1d, gidx, xemb_pad, vert_emb_t, zeros_blk)
    attr_part = part_pre[0]
    emb_part = part_pre[1]
    veb = veb_pad[:_N]

    h0, ea_agg = pl.pallas_call(
        _pre_body,
        out_shape=(
            jax.ShapeDtypeStruct((_N, _H), jnp.float32),
            jax.ShapeDtypeStruct((_N, _H), jnp.float32),
        ),
    )(x, veb, attr_part, emb_part, vert_ff_w, edge_ff_w)

    spmv = _make_spmv()
    layer = pl.pallas_call(
        _layer_body,
        out_shape=jax.ShapeDtypeStruct((_N, _H), jnp.float32),
    )

    h = h0
    for l in range(_L):
        part = spmv(h, src1d, dst1d, zeros_blk)
        h = layer(h, part, ea_agg, edge_lin_w[l], conv_rel_w[l],
                  conv_root_w[l], conv_b[l].reshape(1, _H),
                  bn_gamma[l].reshape(1, _H), bn_beta[l].reshape(1, _H),
                  res_lin_w[l])

    y = pl.pallas_call(
        _head_body,
        out_shape=jax.ShapeDtypeStruct((_N, 1), jnp.float32),
    )(h, head_w1, head_b1.reshape(1, -1), head_w2, head_b2.reshape(1, 1))
    return y


# speedup vs baseline: 1.0768x; 1.0768x over previous
import functools

import jax
import jax.numpy as jnp
from jax import lax
from jax.experimental import pallas as pl
from jax.experimental.pallas import tpu as pltpu
from jax.experimental.pallas import tpu_sc as plsc

NC = 2
NS = 16
NW = NC * NS
LANES = 16

_N = 10000
_E = 320000
_H = 128
_L = 5
_CH = 128
_NCHUNK = _E // _CH
_JMAX = -(-_NCHUNK // NW)
_AROWS = _N
_ZR = 80
_ZNCHUNK = _N // _ZR
_ZQ = -(-_ZNCHUNK // NS)
_NV = 10240
_VNCHUNK = _NV // _CH
_VJ = -(-_VNCHUNK // NW)
_CW = 32
_REP = 64


def _zero_2d(ref, nrows, ncols):
    z = jnp.zeros((LANES,), jnp.float32)

    def body(i, _):
        for g in range(ncols // LANES):
            ref[i, pl.ds(g * LANES, LANES)] = z
        return 0

    lax.fori_loop(0, nrows, body, 0)


def _make_edge_pre():
    mesh = plsc.VectorSubcoreMesh(core_axis_name="c", subcore_axis_name="s")

    @functools.partial(
        pl.kernel,
        out_type=(
            jax.ShapeDtypeStruct((2, NC, _N, _H), jnp.float32),
            jax.ShapeDtypeStruct((_NV, _H), jnp.float32),
        ),
        mesh=mesh,
        scratch_types=[
            pltpu.VMEM_SHARED((_AROWS, _H), jnp.float32),
            pltpu.VMEM((2, _CH, _H), jnp.float32),
            pltpu.VMEM((2, _CH), jnp.int32),
            pltpu.VMEM((2, _CH), jnp.int32),
            pltpu.SemaphoreType.DMA((2,)),
            pltpu.SemaphoreType.DMA((2,)),
            pltpu.SemaphoreType.DMA((2,)),
        ],
    )
    def k(tab_hbm, dst_hbm, gidx_hbm, xemb_hbm, vemb_hbm, zeros_hbm,
          part_out, veb_out,
          acc, rows, gb, db, isem, gsem, ssem):
        core = lax.axis_index("c")
        sid = lax.axis_index("s")
        wid = sid * NC + core
        nj = (_NCHUNK - wid + NW - 1) // NW

        def phase_body(phase, _):
            g0 = phase * _E

            def start_idx(j, p):
                e0 = (wid + NW * j) * _CH
                pltpu.async_copy(gidx_hbm.at[pl.ds(g0 + e0, _CH)],
                                 gb.at[p], isem.at[p])
                pltpu.async_copy(dst_hbm.at[pl.ds(e0, _CH)],
                                 db.at[p], isem.at[p])

            def wait_idx(p):
                pltpu.make_async_copy(gidx_hbm.at[pl.ds(g0, _CH)], gb.at[p],
                                      isem.at[p]).wait()
                pltpu.make_async_copy(dst_hbm.at[pl.ds(0, _CH)], db.at[p],
                                      isem.at[p]).wait()

            def start_gather(p):
                pltpu.async_copy(tab_hbm.at[gb.at[p]], rows.at[p],
                                 gsem.at[p])

            def finish(p):
                pltpu.make_async_copy(tab_hbm.at[gb.at[p]], rows.at[p],
                                      gsem.at[p]).wait()
                pltpu.async_copy(rows.at[p], acc.at[db.at[p]], ssem.at[p],
                                 add=True)

            def wait_scat(p):
                pltpu.make_async_copy(rows.at[p], acc.at[db.at[p]],
                                      ssem.at[p]).wait()

            start_idx(0, 0)
            start_idx(1, 1)

            def zbody(q, _):
                zc = sid + NS * q

                @pl.when(zc < _ZNCHUNK)
                def _():
                    r0 = pl.multiple_of(zc * _ZR, _ZR)
                    pltpu.async_copy(zeros_hbm, acc.at[pl.ds(r0, _ZR)],
                                     ssem.at[0])

                return 0

            def zdrain(q, _):
                zc = sid + NS * q

                @pl.when(zc < _ZNCHUNK)
                def _():
                    r0 = pl.multiple_of(zc * _ZR, _ZR)
                    pltpu.make_async_copy(zeros_hbm, acc.at[pl.ds(r0, _ZR)],
                                          ssem.at[0]).wait()

                return 0

            lax.fori_loop(0, _ZQ, zbody, 0)
            wait_idx(0)
            start_gather(0)
            lax.fori_loop(0, _ZQ, zdrain, 0)
            plsc.subcore_barrier()

            def body(j, _):
                p = lax.rem(j, 2)
                q = 1 - p

                @pl.when(j + 1 < nj)
                def _():
                    wait_idx(q)

                    @pl.when(j >= 1)
                    def _():
                        wait_scat(q)

                    start_gather(q)

                finish(p)

                @pl.when(j + 2 < nj)
                def _():
                    start_idx(j + 2, p)

                return 0

            lax.fori_loop(0, nj, body, 0)
            wait_scat(lax.rem(nj - 1, 2))
            wait_scat(lax.rem(nj, 2))

            plsc.subcore_barrier()

            def cbody(q, _):
                zc = sid + NS * q

                @pl.when(zc < _ZNCHUNK)
                def _():
                    r0 = pl.multiple_of(zc * _ZR, _ZR)
                    bb = rows.at[0, pl.ds(0, _ZR)]
                    pltpu.sync_copy(acc.at[pl.ds(r0, _ZR)], bb)
                    pltpu.sync_copy(bb,
                                    part_out.at[phase, core, pl.ds(r0, _ZR)])

                return 0

            lax.fori_loop(0, _ZQ, cbody, 0)
            plsc.subcore_barrier()
            return 0

        lax.fori_loop(0, 2, phase_body, 0)

        def vbody(j, _):
            cid = wid + NW * j

            @pl.when(cid < _VNCHUNK)
            def _():
                pltpu.sync_copy(xemb_hbm.at[pl.ds(cid * _CH, _CH)],
                                gb.at[0])
                pltpu.async_copy(vemb_hbm.at[gb.at[0]], rows.at[0],
                                 gsem.at[0]).wait()
                pltpu.sync_copy(rows.at[0],
                                veb_out.at[pl.ds(cid * _CH, _CH)])

            return 0

        lax.fori_loop(0, _VJ, vbody, 0)

    return k


def _make_spmv():
    mesh = plsc.VectorSubcoreMesh(core_axis_name="c", subcore_axis_name="s")

    @functools.partial(
        pl.kernel,
        out_type=jax.ShapeDtypeStruct((NC, _N, _H), jnp.float32),
        mesh=mesh,
        scratch_types=[
            pltpu.VMEM_SHARED((_AROWS, _H), jnp.float32),
            pltpu.VMEM((2, _CH, _H), jnp.float32),
            pltpu.VMEM((2, _CH), jnp.int32),
            pltpu.VMEM((2, _CH), jnp.int32),
            pltpu.SemaphoreType.DMA((2,)),
            pltpu.SemaphoreType.DMA((2,)),
            pltpu.SemaphoreType.DMA((2,)),
        ],
    )
    def k(h_hbm, src_hbm, dst_hbm, zeros_hbm, part_out, acc, rows, gb, db,
          isem, gsem, ssem):
        core = lax.axis_index("c")
        sid = lax.axis_index("s")
        wid = sid * NC + core
        nj = (_NCHUNK - wid + NW - 1) // NW

        def start_idx(j, p):
            e0 = (wid + NW * j) * _CH
            pltpu.async_copy(src_hbm.at[pl.ds(e0, _CH)], gb.at[p],
                             isem.at[p])
            pltpu.async_copy(dst_hbm.at[pl.ds(e0, _CH)], db.at[p],
                             isem.at[p])

        def wait_idx(p):
            pltpu.make_async_copy(src_hbm.at[pl.ds(0, _CH)], gb.at[p],
                                  isem.at[p]).wait()
            pltpu.make_async_copy(dst_hbm.at[pl.ds(0, _CH)], db.at[p],
                                  isem.at[p]).wait()

        def start_gather(p):
            pltpu.async_copy(h_hbm.at[gb.at[p]], rows.at[p], gsem.at[p])

        def finish(p):
            pltpu.make_async_copy(h_hbm.at[gb.at[p]], rows.at[p],
                                  gsem.at[p]).wait()
            pltpu.async_copy(rows.at[p], acc.at[db.at[p]], ssem.at[p],
                             add=True)

        def wait_scat(p):
            pltpu.make_async_copy(rows.at[p], acc.at[db.at[p]],
                                  ssem.at[p]).wait()

        start_idx(0, 0)
        start_idx(1, 1)

        def zbody(q, _):
            zc = sid + NS * q

            @pl.when(zc < _ZNCHUNK)
            def _():
                r0 = pl.multiple_of(zc * _ZR, _ZR)
                pltpu.async_copy(zeros_hbm, acc.at[pl.ds(r0, _ZR)],
                                 ssem.at[0])

            return 0

        def zdrain(q, _):
            zc = sid + NS * q

            @pl.when(zc < _ZNCHUNK)
            def _():
                r0 = pl.multiple_of(zc * _ZR, _ZR)
                pltpu.make_async_copy(zeros_hbm, acc.at[pl.ds(r0, _ZR)],
                                      ssem.at[0]).wait()

            return 0

        lax.fori_loop(0, _ZQ, zbody, 0)
        wait_idx(0)
        start_gather(0)
        lax.fori_loop(0, _ZQ, zdrain, 0)
        plsc.subcore_barrier()

        def body(j, _):
            p = lax.rem(j, 2)
            q = 1 - p

            @pl.when(j + 1 < nj)
            def _():
                wait_idx(q)

                @pl.when(j >= 1)
                def _():
                    wait_scat(q)

                start_gather(q)

            finish(p)

            @pl.when(j + 2 < nj)
            def _():
                start_idx(j + 2, p)

            return 0

        lax.fori_loop(0, nj, body, 0)
        wait_scat(lax.rem(nj - 1, 2))
        wait_scat(lax.rem(nj, 2))

        plsc.subcore_barrier()

        def cbody(q, _):
            zc = sid + NS * q

            @pl.when(zc < _ZNCHUNK)
            def _():
                r0 = pl.multiple_of(zc * _ZR, _ZR)
                bb = rows.at[0, pl.ds(0, _ZR)]
                pltpu.sync_copy(acc.at[pl.ds(r0, _ZR)], bb)
                pltpu.sync_copy(bb, part_out.at[core, pl.ds(r0, _ZR)])

            return 0

        lax.fori_loop(0, _ZQ, cbody, 0)

    return k


def _dot(a, b):
    return jnp.dot(a, b, preferred_element_type=jnp.float32)


def _pre_body(x_ref, veb_ref, ap_ref, ep_ref, wv_ref, we_ref,
              h0_ref, eagg_ref):
    h0_ref[...] = _dot(x_ref[...], wv_ref[...]) + veb_ref[...]
    attr = ap_ref[0] + ap_ref[1]
    eagg_ref[...] = _dot(attr, we_ref[...]) + ep_ref[0] + ep_ref[1]


def _layer_common(h_ref, part_ref, eagg_ref, wlin_ref, wr_ref, wroot_ref,
                  b_ref, g_ref, be_ref, wres_ref):
    h = h_ref[...]
    agg = part_ref[0] + part_ref[1] + _dot(eagg_ref[...], wlin_ref[...])
    out = _dot(agg, wr_ref[...]) + _dot(h, wroot_ref[...]) + b_ref[...]
    out = jnp.maximum(out, 0.0)
    mean = jnp.mean(out, axis=0, keepdims=True)
    ctr = out - mean
    var = jnp.mean(ctr * ctr, axis=0, keepdims=True)
    outn = ctr * lax.rsqrt(var + 1e-5) * g_ref[...] + be_ref[...]
    return outn + _dot(h, wres_ref[...])


def _layer_body(h_ref, part_ref, eagg_ref, wlin_ref, wr_ref, wroot_ref,
                b_ref, g_ref, be_ref, wres_ref, out_ref):
    out_ref[...] = _layer_common(h_ref, part_ref, eagg_ref, wlin_ref, wr_ref,
                                 wroot_ref, b_ref, g_ref, be_ref, wres_ref)


def _last_layer_body(h_ref, part_ref, eagg_ref, wlin_ref, wr_ref, wroot_ref,
                     b_ref, g_ref, be_ref, wres_ref, w1_ref, b1_ref, w2_ref,
                     b2_ref, y_ref):
    hf = _layer_common(h_ref, part_ref, eagg_ref, wlin_ref, wr_ref, wroot_ref,
                       b_ref, g_ref, be_ref, wres_ref)
    t = _dot(hf, w1_ref[...]) + b1_ref[...]
    t = t * 0.5 * (1.0 + lax.erf(t * (2.0 ** -0.5)))
    y_ref[...] = _dot(t, w2_ref[...]) + b2_ref[...]


def kernel(x, x_emb, edge_index, edge_attr, edge_attr_emb, vert_ff_w,
           vert_emb_t, edge_ff_w, edge_emb_t, edge_lin_w, conv_rel_w,
           conv_root_w, conv_b, bn_gamma, bn_beta, res_lin_w, head_w1,
           head_b1, head_w2, head_b2):
    evoc = edge_emb_t.shape[0]
    src1d = edge_index[0]
    dst1d = edge_index[1]
    te_rep = jnp.tile(edge_emb_t, (_REP, 1))
    tab = jnp.concatenate([edge_attr, te_rep])
    eae_spread = (edge_attr_emb.astype(jnp.int32)
                  + evoc * (jnp.arange(_E, dtype=jnp.int32) % _REP))
    gidx = jnp.concatenate([
        jnp.arange(_E, dtype=jnp.int32),
        eae_spread + _E,
    ])
    xemb_pad = jnp.concatenate(
        [x_emb.astype(jnp.int32), jnp.zeros((_NV - _N,), jnp.int32)])
    zeros_blk = jnp.zeros((_ZR, _H), jnp.float32)

    part_pre, veb_pad = _make_edge_pre()(
        tab, dst1d, gidx, xemb_pad, vert_emb_t, zeros_blk)
    attr_part = part_pre[0]
    emb_part = part_pre[1]
    veb = veb_pad[:_N]

    h0, ea_agg = pl.pallas_call(
        _pre_body,
        out_shape=(
            jax.ShapeDtypeStruct((_N, _H), jnp.float32),
            jax.ShapeDtypeStruct((_N, _H), jnp.float32),
        ),
    )(x, veb, attr_part, emb_part, vert_ff_w, edge_ff_w)

    spmv = _make_spmv()
    layer = pl.pallas_call(
        _layer_body,
        out_shape=jax.ShapeDtypeStruct((_N, _H), jnp.float32),
    )

    h = h0
    for l in range(_L - 1):
        part = spmv(h, src1d, dst1d, zeros_blk)
        h = layer(h, part, ea_agg, edge_lin_w[l], conv_rel_w[l],
                  conv_root_w[l], conv_b[l].reshape(1, _H),
                  bn_gamma[l].reshape(1, _H), bn_beta[l].reshape(1, _H),
                  res_lin_w[l])

    l = _L - 1
    part = spmv(h, src1d, dst1d, zeros_blk)
    y = pl.pallas_call(
        _last_layer_body,
        out_shape=jax.ShapeDtypeStruct((_N, 1), jnp.float32),
    )(h, part, ea_agg, edge_lin_w[l], conv_rel_w[l], conv_root_w[l],
      conv_b[l].reshape(1, _H), bn_gamma[l].reshape(1, _H),
      bn_beta[l].reshape(1, _H), res_lin_w[l], head_w1,
      head_b1.reshape(1, -1), head_w2, head_b2.reshape(1, 1))
    return y
